# Initial kernel scaffold; baseline (speedup 1.0000x reference)
import jax
import jax.numpy as jnp
from jax.experimental import pallas as pl


def kernel(inputs, state, read_W, read_b, write_W, write_b, kern_W, kern_b, proj_W, proj_b, ln_gamma, ln_beta, readout_W, readout_b):
    raise NotImplementedError("write your pallas kernel here")


# speedup vs baseline: 2.1031x; 2.1031x over previous
import jax
import jax.numpy as jnp
from jax.experimental import pallas as pl
from jax.experimental.pallas import tpu as pltpu

_UNITS = 1024
_MEMSIZE = 32
_NUMHEADS = 16
_HEADSIZE = _UNITS // _NUMHEADS
_DEPTH = 8
_EPS = 1e-3
_BB = 16


def _softmax_m(logits):
    mx = jnp.max(logits, axis=1, keepdims=True)
    e = jnp.exp(logits - mx)
    return e / jnp.sum(e, axis=1, keepdims=True)


def _expand_heads(w):
    b, m, h = w.shape
    return jnp.broadcast_to(w[..., None], (b, m, h, _HEADSIZE)).reshape(
        b, m, h * _HEADSIZE)


def _dnc_kernel(inputs_ref, state_ref, read_w_ref, read_b_ref, write_w_ref,
                write_b_ref, kern_w_ref, kern_b_ref, proj_w_ref, proj_b_ref,
                ln_g_ref, ln_b_ref, ro_w_ref, ro_b_ref,
                out_ref, memout_ref, mem_ref):
    l = pl.program_id(1)
    bb = mem_ref.shape[0]

    @pl.when(l == 0)
    def _init():
        comb = inputs_ref[...] + state_ref[...]
        m4 = comb.reshape(bb, _NUMHEADS, _MEMSIZE, _HEADSIZE)
        mem_ref[...] = m4.transpose(0, 2, 1, 3).reshape(bb, _MEMSIZE, _UNITS)

    @pl.when(l == _DEPTH // 2)
    def _mid():
        for p in range(2):
            half = mem_ref[:, p * 16:(p + 1) * 16, :]
            h4 = half.reshape(bb, 16, _NUMHEADS, _HEADSIZE)
            mem_ref[:, p * 16:(p + 1) * 16, :] = \
                h4.transpose(0, 2, 1, 3).reshape(bb, 16, _UNITS)

    mem = mem_ref[...]
    mean = jnp.mean(mem, axis=1, keepdims=True)
    keys = mem + mean
    keys2 = keys.reshape(bb * _MEMSIZE, _UNITS)

    logits = jnp.dot(keys2, read_w_ref[0],
                     preferred_element_type=jnp.float32)
    logits = logits.reshape(bb, _MEMSIZE, _NUMHEADS) + read_b_ref[0]
    w = _expand_heads(_softmax_m(logits))
    att = jnp.sum(w * mem, axis=1)

    v = jnp.maximum(
        jnp.dot(att, kern_w_ref[0], preferred_element_type=jnp.float32)
        + kern_b_ref[0], 0.0)
    v = jnp.dot(v, proj_w_ref[0], preferred_element_type=jnp.float32) \
        + proj_b_ref[0]
    mu = jnp.mean(v, axis=-1, keepdims=True)
    var = jnp.mean(jnp.square(v - mu), axis=-1, keepdims=True)
    v = (v - mu) * jax.lax.rsqrt(var + _EPS) * ln_g_ref[...] + ln_b_ref[...]

    wl = jnp.dot((keys + v[:, None, :]).reshape(bb * _MEMSIZE, _UNITS),
                 write_w_ref[0], preferred_element_type=jnp.float32)
    wl = wl.reshape(bb, _MEMSIZE, _NUMHEADS) + write_b_ref[0]
    ww = _expand_heads(_softmax_m(wl))
    newmem = (1.0 - ww) * mem + ww * v[:, None, :]
    mem_ref[...] = newmem

    @pl.when(l == _DEPTH - 1)
    def _final():
        mean = jnp.mean(newmem, axis=1, keepdims=True)
        keys = (newmem + mean).reshape(bb * _MEMSIZE, _UNITS)
        logits = jnp.dot(keys, ro_w_ref[...],
                         preferred_element_type=jnp.float32)
        logits = logits.reshape(bb, _MEMSIZE, _NUMHEADS) + ro_b_ref[...]
        w = _expand_heads(_softmax_m(logits))
        out_ref[...] = jnp.sum(w * newmem, axis=1)
        for slot in range(_MEMSIZE):
            memout_ref[:, slot, :] = newmem[:, (slot % 2) * 16 + slot // 2, :]


def kernel(inputs, state, read_W, read_b, write_W, write_b, kern_W, kern_b,
           proj_W, proj_b, ln_gamma, ln_beta, readout_W, readout_b):
    B = inputs.shape[0]
    nb = B // _BB

    read_b3 = read_b.reshape(_DEPTH, 1, _NUMHEADS)
    write_b3 = write_b.reshape(_DEPTH, 1, _NUMHEADS)
    kern_b3 = kern_b.reshape(_DEPTH, 1, _UNITS)
    proj_b3 = proj_b.reshape(_DEPTH, 1, _UNITS)
    ln_g2 = ln_gamma.reshape(1, _UNITS)
    ln_b2 = ln_beta.reshape(1, _UNITS)
    ro_b2 = readout_b.reshape(1, _NUMHEADS)

    bi = lambda b, l: (b, 0)
    li = lambda b, l: (l, 0, 0)
    fixed2 = lambda b, l: (0, 0)

    out, memout = pl.pallas_call(
        _dnc_kernel,
        grid=(nb, _DEPTH),
        in_specs=[
            pl.BlockSpec((_BB, _MEMSIZE * _UNITS), bi),
            pl.BlockSpec((_BB, _MEMSIZE * _UNITS), bi),
            pl.BlockSpec((1, _UNITS, _NUMHEADS), li),
            pl.BlockSpec((1, 1, _NUMHEADS), li),
            pl.BlockSpec((1, _UNITS, _NUMHEADS), li),
            pl.BlockSpec((1, 1, _NUMHEADS), li),
            pl.BlockSpec((1, _UNITS, _UNITS), li),
            pl.BlockSpec((1, 1, _UNITS), li),
            pl.BlockSpec((1, _UNITS, _UNITS), li),
            pl.BlockSpec((1, 1, _UNITS), li),
            pl.BlockSpec((1, _UNITS), fixed2),
            pl.BlockSpec((1, _UNITS), fixed2),
            pl.BlockSpec((_UNITS, _NUMHEADS), fixed2),
            pl.BlockSpec((1, _NUMHEADS), fixed2),
        ],
        out_specs=[
            pl.BlockSpec((_BB, _UNITS), bi),
            pl.BlockSpec((_BB, _MEMSIZE, _UNITS), lambda b, l: (b, 0, 0)),
        ],
        out_shape=[
            jax.ShapeDtypeStruct((B, _UNITS), jnp.float32),
            jax.ShapeDtypeStruct((B, _MEMSIZE, _UNITS), jnp.float32),
        ],
        scratch_shapes=[pltpu.VMEM((_BB, _MEMSIZE, _UNITS), jnp.float32)],
        compiler_params=pltpu.CompilerParams(
            dimension_semantics=("parallel", "arbitrary"),
            vmem_limit_bytes=56 * 1024 * 1024,
        ),
        name="dnccell",
    )(inputs, state, read_W, read_b3, write_W, write_b3, kern_W, kern_b3,
      proj_W, proj_b3, ln_g2, ln_b2, readout_W, ro_b2)
    return out, memout.reshape(B, _MEMSIZE * _UNITS)


# speedup vs baseline: 3.3940x; 1.6138x over previous
import jax
import jax.numpy as jnp
from jax.experimental import pallas as pl
from jax.experimental.pallas import tpu as pltpu

_UNITS = 1024
_MEMSIZE = 32
_NUMHEADS = 16
_HEADSIZE = _UNITS // _NUMHEADS
_DEPTH = 8
_EPS = 1e-3
_BB = 16


def _softmax_m(logits):
    mx = jnp.max(logits, axis=1, keepdims=True)
    e = jnp.exp(logits - mx)
    return e / jnp.sum(e, axis=1, keepdims=True)


def _expand_heads(w, eye_ref):
    b, m, h = w.shape
    we = jnp.dot(w.reshape(b * m, h), eye_ref[...],
                 preferred_element_type=jnp.float32)
    return we.reshape(b, m, _UNITS)


def _read_att(mem2d, mem3, mean, w_ref, b_ref, eye_ref):
    bb = mem3.shape[0]
    logits = jnp.dot(mem2d, w_ref, preferred_element_type=jnp.float32)
    corr = jnp.dot(mean, w_ref, preferred_element_type=jnp.float32) + b_ref
    logits = logits.reshape(bb, _MEMSIZE, _NUMHEADS) + corr[:, None, :]
    return _expand_heads(_softmax_m(logits), eye_ref)


def _dnc_kernel(inputs_ref, state_ref, eye_ref, read_w_ref, read_b_ref,
                write_w_ref, write_b_ref, kern_w_ref, kern_b_ref, proj_w_ref,
                proj_b_ref, ln_g_ref, ln_b_ref, ro_w_ref, ro_b_ref,
                out_ref, memout_ref, mem_ref):
    l = pl.program_id(1)
    bb = mem_ref.shape[0]

    @pl.when(l == 0)
    def _init():
        comb = inputs_ref[...] + state_ref[...]
        c4 = comb.reshape(bb, _NUMHEADS, _MEMSIZE, _HEADSIZE)
        mem_ref[...] = jnp.concatenate(
            [c4[:, h] for h in range(_NUMHEADS)], axis=-1)

    @pl.when(l == _DEPTH // 2)
    def _mid():
        for p in range(2):
            h4 = mem_ref[:, p * 16:(p + 1) * 16, :].reshape(
                bb, 16, _NUMHEADS, _HEADSIZE)
            mem_ref[:, p * 16:(p + 1) * 16, :] = jnp.concatenate(
                [h4[:, h] for h in range(_NUMHEADS)], axis=-1)

    mem = mem_ref[...]
    mem2d = mem.reshape(bb * _MEMSIZE, _UNITS)
    mean = jnp.mean(mem, axis=1)

    w = _read_att(mem2d, mem, mean, read_w_ref[0], read_b_ref[0],
                  eye_ref)
    att = jnp.sum(w * mem, axis=1)

    v = jnp.maximum(
        jnp.dot(att, kern_w_ref[0], preferred_element_type=jnp.float32)
        + kern_b_ref[0], 0.0)
    v = jnp.dot(v, proj_w_ref[0], preferred_element_type=jnp.float32) \
        + proj_b_ref[0]
    mu = jnp.mean(v, axis=-1, keepdims=True)
    var = jnp.mean(jnp.square(v - mu), axis=-1, keepdims=True)
    v = (v - mu) * jax.lax.rsqrt(var + _EPS) * ln_g_ref[...] + ln_b_ref[...]

    wl = jnp.dot(mem2d, write_w_ref[0], preferred_element_type=jnp.float32)
    wcorr = jnp.dot(mean + v, write_w_ref[0],
                    preferred_element_type=jnp.float32) + write_b_ref[0]
    wl = wl.reshape(bb, _MEMSIZE, _NUMHEADS) + wcorr[:, None, :]
    ww = _expand_heads(_softmax_m(wl), eye_ref)
    newmem = (1.0 - ww) * mem + ww * v[:, None, :]
    mem_ref[...] = newmem

    @pl.when(l == _DEPTH - 1)
    def _final():
        nm2d = newmem.reshape(bb * _MEMSIZE, _UNITS)
        nmean = jnp.mean(newmem, axis=1)
        w = _read_att(nm2d, newmem, nmean, ro_w_ref[...], ro_b_ref[...],
                      eye_ref)
        out_ref[...] = jnp.sum(w * newmem, axis=1)
        for slot in range(_MEMSIZE):
            memout_ref[:, slot, :] = newmem[:, (slot % 2) * 16 + slot // 2, :]


def kernel(inputs, state, read_W, read_b, write_W, write_b, kern_W, kern_b,
           proj_W, proj_b, ln_gamma, ln_beta, readout_W, readout_b):
    B = inputs.shape[0]
    nb = B // _BB

    read_b3 = read_b.reshape(_DEPTH, 1, _NUMHEADS)
    write_b3 = write_b.reshape(_DEPTH, 1, _NUMHEADS)
    kern_b3 = kern_b.reshape(_DEPTH, 1, _UNITS)
    proj_b3 = proj_b.reshape(_DEPTH, 1, _UNITS)
    ln_g2 = ln_gamma.reshape(1, _UNITS)
    ln_b2 = ln_beta.reshape(1, _UNITS)
    ro_b2 = readout_b.reshape(1, _NUMHEADS)
    eye = jnp.equal(
        jax.lax.broadcasted_iota(jnp.int32, (_NUMHEADS, _UNITS), 1)
        // _HEADSIZE,
        jax.lax.broadcasted_iota(jnp.int32, (_NUMHEADS, _UNITS), 0)
    ).astype(jnp.float32)

    bi = lambda b, l: (b, 0)
    li = lambda b, l: (l, 0, 0)
    fixed2 = lambda b, l: (0, 0)

    out, memout = pl.pallas_call(
        _dnc_kernel,
        grid=(nb, _DEPTH),
        in_specs=[
            pl.BlockSpec((_BB, _MEMSIZE * _UNITS), bi),
            pl.BlockSpec((_BB, _MEMSIZE * _UNITS), bi),
            pl.BlockSpec((_NUMHEADS, _UNITS), fixed2),
            pl.BlockSpec((1, _UNITS, _NUMHEADS), li),
            pl.BlockSpec((1, 1, _NUMHEADS), li),
            pl.BlockSpec((1, _UNITS, _NUMHEADS), li),
            pl.BlockSpec((1, 1, _NUMHEADS), li),
            pl.BlockSpec((1, _UNITS, _UNITS), li),
            pl.BlockSpec((1, 1, _UNITS), li),
            pl.BlockSpec((1, _UNITS, _UNITS), li),
            pl.BlockSpec((1, 1, _UNITS), li),
            pl.BlockSpec((1, _UNITS), fixed2),
            pl.BlockSpec((1, _UNITS), fixed2),
            pl.BlockSpec((_UNITS, _NUMHEADS), fixed2),
            pl.BlockSpec((1, _NUMHEADS), fixed2),
        ],
        out_specs=[
            pl.BlockSpec((_BB, _UNITS), bi),
            pl.BlockSpec((_BB, _MEMSIZE, _UNITS), lambda b, l: (b, 0, 0)),
        ],
        out_shape=[
            jax.ShapeDtypeStruct((B, _UNITS), jnp.float32),
            jax.ShapeDtypeStruct((B, _MEMSIZE, _UNITS), jnp.float32),
        ],
        scratch_shapes=[pltpu.VMEM((_BB, _MEMSIZE, _UNITS), jnp.float32)],
        compiler_params=pltpu.CompilerParams(
            dimension_semantics=("parallel", "arbitrary"),
            vmem_limit_bytes=56 * 1024 * 1024,
        ),
        name="dnccell",
    )(inputs, state, eye, read_W, read_b3, write_W, write_b3, kern_W,
      kern_b3, proj_W, proj_b3, ln_g2, ln_b2, readout_W, ro_b2)
    return out, memout.reshape(B, _MEMSIZE * _UNITS)


# speedup vs baseline: 3.6900x; 1.0872x over previous
import jax
import jax.numpy as jnp
from jax.experimental import pallas as pl
from jax.experimental.pallas import tpu as pltpu

_UNITS = 1024
_MEMSIZE = 32
_NUMHEADS = 16
_HEADSIZE = _UNITS // _NUMHEADS
_DEPTH = 8
_EPS = 1e-3
_BB = 16


def _softmax_m(logits):
    mx = jnp.max(logits, axis=1, keepdims=True)
    e = jnp.exp(logits - mx)
    return e / jnp.sum(e, axis=1, keepdims=True)


def _expand_heads(w, eye_ref):
    b, m, h = w.shape
    we = jnp.dot(w.reshape(b * m, h), eye_ref[...],
                 preferred_element_type=jnp.float32)
    return we.reshape(b, m, _UNITS)


def _read_att(mem2d, mem3, mean, w_ref, b_ref, eye_ref):
    bb = mem3.shape[0]
    logits = jnp.dot(mem2d, w_ref, preferred_element_type=jnp.float32)
    corr = jnp.dot(mean, w_ref, preferred_element_type=jnp.float32) + b_ref
    logits = logits.reshape(bb, _MEMSIZE, _NUMHEADS) + corr[:, None, :]
    return _expand_heads(_softmax_m(logits), eye_ref)


def _dnc_kernel(inputs_ref, state_ref, eye_ref, read_w_ref, read_b_ref,
                write_w_ref, write_b_ref, kern_w_ref, kern_b_ref, proj_w_ref,
                proj_b_ref, ln_g_ref, ln_b_ref, ro_w_ref, ro_b_ref,
                out_ref, memout_ref, mem_all_ref):
    l = pl.program_id(1)
    i = pl.program_id(2)
    mem_ref = mem_all_ref.at[i]
    bb = _BB

    @pl.when(l == 0)
    def _init():
        comb = inputs_ref[...] + state_ref[...]
        c4 = comb.reshape(bb, _NUMHEADS, _MEMSIZE, _HEADSIZE)
        mem_ref[...] = jnp.concatenate(
            [c4[:, h] for h in range(_NUMHEADS)], axis=-1)

    @pl.when(l == _DEPTH // 2)
    def _mid():
        for p in range(2):
            h4 = mem_ref[:, p * 16:(p + 1) * 16, :].reshape(
                bb, 16, _NUMHEADS, _HEADSIZE)
            mem_ref[:, p * 16:(p + 1) * 16, :] = jnp.concatenate(
                [h4[:, h] for h in range(_NUMHEADS)], axis=-1)

    mem = mem_ref[...]
    mem2d = mem.reshape(bb * _MEMSIZE, _UNITS)
    mean = jnp.mean(mem, axis=1)

    w = _read_att(mem2d, mem, mean, read_w_ref[0], read_b_ref[0],
                  eye_ref)
    att = jnp.sum(w * mem, axis=1)

    v = jnp.maximum(
        jnp.dot(att, kern_w_ref[0], preferred_element_type=jnp.float32)
        + kern_b_ref[0], 0.0)
    v = jnp.dot(v, proj_w_ref[0], preferred_element_type=jnp.float32) \
        + proj_b_ref[0]
    mu = jnp.mean(v, axis=-1, keepdims=True)
    var = jnp.mean(jnp.square(v - mu), axis=-1, keepdims=True)
    v = (v - mu) * jax.lax.rsqrt(var + _EPS) * ln_g_ref[...] + ln_b_ref[...]

    wl = jnp.dot(mem2d, write_w_ref[0], preferred_element_type=jnp.float32)
    wcorr = jnp.dot(mean + v, write_w_ref[0],
                    preferred_element_type=jnp.float32) + write_b_ref[0]
    wl = wl.reshape(bb, _MEMSIZE, _NUMHEADS) + wcorr[:, None, :]
    ww = _expand_heads(_softmax_m(wl), eye_ref)
    newmem = (1.0 - ww) * mem + ww * v[:, None, :]
    mem_ref[...] = newmem

    @pl.when(l == _DEPTH - 1)
    def _final():
        nm2d = newmem.reshape(bb * _MEMSIZE, _UNITS)
        nmean = jnp.mean(newmem, axis=1)
        w = _read_att(nm2d, newmem, nmean, ro_w_ref[...], ro_b_ref[...],
                      eye_ref)
        out_ref[...] = jnp.sum(w * newmem, axis=1)
        for slot in range(_MEMSIZE):
            memout_ref[:, slot, :] = newmem[:, (slot % 2) * 16 + slot // 2, :]


_NI = 8
_NO = 4


def kernel(inputs, state, read_W, read_b, write_W, write_b, kern_W, kern_b,
           proj_W, proj_b, ln_gamma, ln_beta, readout_W, readout_b):
    B = inputs.shape[0]

    read_b3 = read_b.reshape(_DEPTH, 1, _NUMHEADS)
    write_b3 = write_b.reshape(_DEPTH, 1, _NUMHEADS)
    kern_b3 = kern_b.reshape(_DEPTH, 1, _UNITS)
    proj_b3 = proj_b.reshape(_DEPTH, 1, _UNITS)
    ln_g2 = ln_gamma.reshape(1, _UNITS)
    ln_b2 = ln_beta.reshape(1, _UNITS)
    ro_b2 = readout_b.reshape(1, _NUMHEADS)
    eye = jnp.equal(
        jax.lax.broadcasted_iota(jnp.int32, (_NUMHEADS, _UNITS), 1)
        // _HEADSIZE,
        jax.lax.broadcasted_iota(jnp.int32, (_NUMHEADS, _UNITS), 0)
    ).astype(jnp.float32)

    bin_ = lambda o, l, i: (o * _NI + jnp.where(l == 0, i, _NI - 1), 0)
    bout2 = lambda o, l, i: (o * _NI + jnp.where(l == _DEPTH - 1, i, 0), 0)
    bout3 = lambda o, l, i: (o * _NI + jnp.where(l == _DEPTH - 1, i, 0), 0, 0)
    li = lambda o, l, i: (l, 0, 0)
    fixed2 = lambda o, l, i: (0, 0)

    out, memout = pl.pallas_call(
        _dnc_kernel,
        grid=(_NO, _DEPTH, _NI),
        in_specs=[
            pl.BlockSpec((_BB, _MEMSIZE * _UNITS), bin_),
            pl.BlockSpec((_BB, _MEMSIZE * _UNITS), bin_),
            pl.BlockSpec((_NUMHEADS, _UNITS), fixed2),
            pl.BlockSpec((1, _UNITS, _NUMHEADS), li),
            pl.BlockSpec((1, 1, _NUMHEADS), li),
            pl.BlockSpec((1, _UNITS, _NUMHEADS), li),
            pl.BlockSpec((1, 1, _NUMHEADS), li),
            pl.BlockSpec((1, _UNITS, _UNITS), li),
            pl.BlockSpec((1, 1, _UNITS), li),
            pl.BlockSpec((1, _UNITS, _UNITS), li),
            pl.BlockSpec((1, 1, _UNITS), li),
            pl.BlockSpec((1, _UNITS), fixed2),
            pl.BlockSpec((1, _UNITS), fixed2),
            pl.BlockSpec((_UNITS, _NUMHEADS), fixed2),
            pl.BlockSpec((1, _NUMHEADS), fixed2),
        ],
        out_specs=[
            pl.BlockSpec((_BB, _UNITS), bout2),
            pl.BlockSpec((_BB, _MEMSIZE, _UNITS), bout3),
        ],
        out_shape=[
            jax.ShapeDtypeStruct((B, _UNITS), jnp.float32),
            jax.ShapeDtypeStruct((B, _MEMSIZE, _UNITS), jnp.float32),
        ],
        scratch_shapes=[
            pltpu.VMEM((_NI, _BB, _MEMSIZE, _UNITS), jnp.float32)],
        compiler_params=pltpu.CompilerParams(
            dimension_semantics=("parallel", "arbitrary", "arbitrary"),
            vmem_limit_bytes=56 * 1024 * 1024,
        ),
        name="dnccell",
    )(inputs, state, eye, read_W, read_b3, write_W, write_b3, kern_W,
      kern_b3, proj_W, proj_b3, ln_g2, ln_b2, readout_W, ro_b2)
    return out, memout.reshape(B, _MEMSIZE * _UNITS)


# speedup vs baseline: 3.7159x; 1.0070x over previous
import jax
import jax.numpy as jnp
from jax.experimental import pallas as pl
from jax.experimental.pallas import tpu as pltpu

_UNITS = 1024
_MEMSIZE = 32
_NUMHEADS = 16
_HEADSIZE = _UNITS // _NUMHEADS
_DEPTH = 8
_EPS = 1e-3
_BB = 16


def _softmax_m(logits):
    mx = jnp.max(logits, axis=1, keepdims=True)
    e = jnp.exp(logits - mx)
    return e / jnp.sum(e, axis=1, keepdims=True)


def _expand_heads(w, eye_ref):
    b, m, h = w.shape
    we = jnp.dot(w.reshape(b * m, h), eye_ref[...],
                 preferred_element_type=jnp.float32)
    return we.reshape(b, m, _UNITS)


def _read_att(mem2d, mem3, mean, w_ref, b_ref, eye_ref):
    bb = mem3.shape[0]
    logits = jnp.dot(mem2d, w_ref, preferred_element_type=jnp.float32)
    corr = jnp.dot(mean, w_ref, preferred_element_type=jnp.float32) + b_ref
    logits = logits.reshape(bb, _MEMSIZE, _NUMHEADS) + corr[:, None, :]
    return _expand_heads(_softmax_m(logits), eye_ref)


def _dnc_kernel(inputs_ref, state_ref, eye_ref, read_w_ref, read_b_ref,
                write_w_ref, write_b_ref, kern_w_ref, kern_b_ref, proj_w_ref,
                proj_b_ref, ln_g_ref, ln_b_ref, ro_w_ref, ro_b_ref,
                out_ref, memout_ref, mem_all_ref, kern_buf, proj_buf,
                sem_k, sem_p):
    l = pl.program_id(1)
    i = pl.program_id(2)
    mem_ref = mem_all_ref.at[i]
    bb = _BB
    slot = jax.lax.rem(l, 2)

    @pl.when(i == 0)
    def _weights():
        @pl.when(l == 0)
        def _():
            pltpu.make_async_copy(kern_w_ref.at[0], kern_buf.at[0],
                                  sem_k.at[0]).start()
            pltpu.make_async_copy(proj_w_ref.at[0], proj_buf.at[0],
                                  sem_p.at[0]).start()
        pltpu.make_async_copy(kern_w_ref.at[l], kern_buf.at[slot],
                              sem_k.at[slot]).wait()
        pltpu.make_async_copy(proj_w_ref.at[l], proj_buf.at[slot],
                              sem_p.at[slot]).wait()

        @pl.when(l < _DEPTH - 1)
        def _():
            nslot = 1 - slot
            pltpu.make_async_copy(kern_w_ref.at[l + 1], kern_buf.at[nslot],
                                  sem_k.at[nslot]).start()
            pltpu.make_async_copy(proj_w_ref.at[l + 1], proj_buf.at[nslot],
                                  sem_p.at[nslot]).start()

    @pl.when(l == 0)
    def _init():
        comb = inputs_ref[...] + state_ref[...]
        c4 = comb.reshape(bb, _NUMHEADS, _MEMSIZE, _HEADSIZE)
        mem_ref[...] = jnp.concatenate(
            [c4[:, h] for h in range(_NUMHEADS)], axis=-1)

    @pl.when(l == _DEPTH // 2)
    def _mid():
        for p in range(2):
            h4 = mem_ref[:, p * 16:(p + 1) * 16, :].reshape(
                bb, 16, _NUMHEADS, _HEADSIZE)
            mem_ref[:, p * 16:(p + 1) * 16, :] = jnp.concatenate(
                [h4[:, h] for h in range(_NUMHEADS)], axis=-1)

    mem = mem_ref[...]
    mem2d = mem.reshape(bb * _MEMSIZE, _UNITS)
    mean = jnp.mean(mem, axis=1)

    w = _read_att(mem2d, mem, mean, read_w_ref[0], read_b_ref[0],
                  eye_ref)
    att = jnp.sum(w * mem, axis=1)

    v = jnp.maximum(
        jnp.dot(att, kern_buf[slot], preferred_element_type=jnp.float32)
        + kern_b_ref[0], 0.0)
    v = jnp.dot(v, proj_buf[slot], preferred_element_type=jnp.float32) \
        + proj_b_ref[0]
    mu = jnp.mean(v, axis=-1, keepdims=True)
    var = jnp.mean(jnp.square(v - mu), axis=-1, keepdims=True)
    v = (v - mu) * jax.lax.rsqrt(var + _EPS) * ln_g_ref[...] + ln_b_ref[...]

    wl = jnp.dot(mem2d, write_w_ref[0], preferred_element_type=jnp.float32)
    wcorr = jnp.dot(mean + v, write_w_ref[0],
                    preferred_element_type=jnp.float32) + write_b_ref[0]
    wl = wl.reshape(bb, _MEMSIZE, _NUMHEADS) + wcorr[:, None, :]
    ww = _expand_heads(_softmax_m(wl), eye_ref)
    newmem = (1.0 - ww) * mem + ww * v[:, None, :]
    mem_ref[...] = newmem

    @pl.when(l == _DEPTH - 1)
    def _final():
        nm2d = newmem.reshape(bb * _MEMSIZE, _UNITS)
        nmean = jnp.mean(newmem, axis=1)
        w = _read_att(nm2d, newmem, nmean, ro_w_ref[...], ro_b_ref[...],
                      eye_ref)
        out_ref[...] = jnp.sum(w * newmem, axis=1)
        for slot in range(_MEMSIZE):
            memout_ref[:, slot, :] = newmem[:, (slot % 2) * 16 + slot // 2, :]


_NI = 8
_NO = 4


def kernel(inputs, state, read_W, read_b, write_W, write_b, kern_W, kern_b,
           proj_W, proj_b, ln_gamma, ln_beta, readout_W, readout_b):
    B = inputs.shape[0]

    read_b3 = read_b.reshape(_DEPTH, 1, _NUMHEADS)
    write_b3 = write_b.reshape(_DEPTH, 1, _NUMHEADS)
    kern_b3 = kern_b.reshape(_DEPTH, 1, _UNITS)
    proj_b3 = proj_b.reshape(_DEPTH, 1, _UNITS)
    ln_g2 = ln_gamma.reshape(1, _UNITS)
    ln_b2 = ln_beta.reshape(1, _UNITS)
    ro_b2 = readout_b.reshape(1, _NUMHEADS)
    eye = jnp.equal(
        jax.lax.broadcasted_iota(jnp.int32, (_NUMHEADS, _UNITS), 1)
        // _HEADSIZE,
        jax.lax.broadcasted_iota(jnp.int32, (_NUMHEADS, _UNITS), 0)
    ).astype(jnp.float32)

    bin_ = lambda o, l, i: (o * _NI + jnp.where(l == 0, i, _NI - 1), 0)
    bout2 = lambda o, l, i: (o * _NI + jnp.where(l == _DEPTH - 1, i, 0), 0)
    bout3 = lambda o, l, i: (o * _NI + jnp.where(l == _DEPTH - 1, i, 0), 0, 0)
    li = lambda o, l, i: (l, 0, 0)
    fixed2 = lambda o, l, i: (0, 0)

    out, memout = pl.pallas_call(
        _dnc_kernel,
        grid=(_NO, _DEPTH, _NI),
        in_specs=[
            pl.BlockSpec((_BB, _MEMSIZE * _UNITS), bin_),
            pl.BlockSpec((_BB, _MEMSIZE * _UNITS), bin_),
            pl.BlockSpec((_NUMHEADS, _UNITS), fixed2),
            pl.BlockSpec((1, _UNITS, _NUMHEADS), li),
            pl.BlockSpec((1, 1, _NUMHEADS), li),
            pl.BlockSpec((1, _UNITS, _NUMHEADS), li),
            pl.BlockSpec((1, 1, _NUMHEADS), li),
            pl.BlockSpec(memory_space=pl.ANY),
            pl.BlockSpec((1, 1, _UNITS), li),
            pl.BlockSpec(memory_space=pl.ANY),
            pl.BlockSpec((1, 1, _UNITS), li),
            pl.BlockSpec((1, _UNITS), fixed2),
            pl.BlockSpec((1, _UNITS), fixed2),
            pl.BlockSpec((_UNITS, _NUMHEADS), fixed2),
            pl.BlockSpec((1, _NUMHEADS), fixed2),
        ],
        out_specs=[
            pl.BlockSpec((_BB, _UNITS), bout2),
            pl.BlockSpec((_BB, _MEMSIZE, _UNITS), bout3),
        ],
        out_shape=[
            jax.ShapeDtypeStruct((B, _UNITS), jnp.float32),
            jax.ShapeDtypeStruct((B, _MEMSIZE, _UNITS), jnp.float32),
        ],
        scratch_shapes=[
            pltpu.VMEM((_NI, _BB, _MEMSIZE, _UNITS), jnp.float32),
            pltpu.VMEM((2, _UNITS, _UNITS), jnp.float32),
            pltpu.VMEM((2, _UNITS, _UNITS), jnp.float32),
            pltpu.SemaphoreType.DMA((2,)),
            pltpu.SemaphoreType.DMA((2,)),
        ],
        compiler_params=pltpu.CompilerParams(
            dimension_semantics=("parallel", "arbitrary", "arbitrary"),
            vmem_limit_bytes=56 * 1024 * 1024,
        ),
        name="dnccell",
    )(inputs, state, eye, read_W, read_b3, write_W, write_b3, kern_W,
      kern_b3, proj_W, proj_b3, ln_g2, ln_b2, readout_W, ro_b2)
    return out, memout.reshape(B, _MEMSIZE * _UNITS)
